# baseline (speedup 1.0000x reference)
import functools

import jax
import jax.numpy as jnp
from jax import lax
from jax.experimental import pallas as pl
from jax.experimental.pallas import tpu as pltpu
from jax.experimental.pallas import tpu_sc as plsc

N_X = 4096
N_T = 4096
LANES = 128
SUB = N_T // LANES
R_TOTAL = N_X * SUB
BLK_X = 128
BLK_R = BLK_X * SUB

_SC_INFO = plsc.get_sparse_core_info()
_NC = _SC_INFO.num_cores
_NS = _SC_INFO.num_subcores
_NW = _NC * _NS
WORDS_PER_W = N_X * N_T // _NW
CHUNK_W = 65536
REP = CHUNK_W // N_T


def _bitonic_sort_2d(a):
    r_iota = jax.lax.broadcasted_iota(jnp.int32, (SUB, LANES), 0)
    c_iota = jax.lax.broadcasted_iota(jnp.int32, (SUB, LANES), 1)
    idx = r_iota * LANES + c_iota
    n = SUB * LANES
    k = 2
    while k <= n:
        j = k // 2
        while j >= 1:
            if j < LANES:
                fwd = jnp.roll(a, -j, axis=1)
                bwd = jnp.roll(a, j, axis=1)
            else:
                jr = j // LANES
                fwd = jnp.roll(a, -jr, axis=0)
                bwd = jnp.roll(a, jr, axis=0)
            lower = (idx & j) == 0
            p = jnp.where(lower, fwd, bwd)
            asc = (idx & k) == 0
            keep_min = lower == asc
            a = jnp.where(keep_min, jnp.minimum(a, p), jnp.maximum(a, p))
            j //= 2
        k *= 2
    return a


def _sort_kernel(t2d, ts_ref):
    ts_ref[:] = _bitonic_sort_2d(t2d[:])


def _xx_kernel(x_col, xx_ref):
    xb = x_col[:].reshape(BLK_X, 1, 1)
    xx_ref[:] = jnp.broadcast_to(xb, (BLK_X, SUB, LANES)).reshape(BLK_R, LANES)


_SC_MESH = plsc.VectorSubcoreMesh(core_axis_name="c", subcore_axis_name="s")


@functools.partial(
    pl.kernel,
    mesh=_SC_MESH,
    out_type=jax.ShapeDtypeStruct((N_X * N_T,), jnp.float32),
    scratch_types=[
        pltpu.VMEM((CHUNK_W,), jnp.float32),
        pltpu.SemaphoreType.DMA,
        pltpu.SemaphoreType.DMA,
    ],
)
def _tt_sc_kernel(ts_hbm, out_hbm, buf, sem_r, sem_w):
    wid = lax.axis_index("s") * _NC + lax.axis_index("c")
    base = wid * WORDS_PER_W
    reads = [
        pltpu.async_copy(ts_hbm, buf.at[pl.ds(r * N_T, N_T)], sem_r)
        for r in range(REP)
    ]
    for cp in reads:
        cp.wait()
    writes = [
        pltpu.async_copy(buf, out_hbm.at[pl.ds(base + ch * CHUNK_W, CHUNK_W)], sem_w)
        for ch in range(WORDS_PER_W // CHUNK_W)
    ]
    for cp in writes:
        cp.wait()


@jax.jit
def kernel(x_grid, t_grid):
    x_col = x_grid.reshape(N_X, 1)
    t2d = t_grid.reshape(SUB, LANES)

    ts2d = pl.pallas_call(
        _sort_kernel,
        out_shape=jax.ShapeDtypeStruct((SUB, LANES), jnp.float32),
    )(t2d)

    tt = _tt_sc_kernel(ts2d.reshape(N_T))

    xx = pl.pallas_call(
        _xx_kernel,
        grid=(N_X // BLK_X,),
        in_specs=[pl.BlockSpec((BLK_X, 1), lambda i: (i, 0))],
        out_specs=pl.BlockSpec((BLK_R, LANES), lambda i: (i, 0)),
        out_shape=jax.ShapeDtypeStruct((R_TOTAL, LANES), jnp.float32),
    )(x_col)

    return (xx.reshape(-1, 1), tt.reshape(-1, 1))


# speedup vs baseline: 1.3239x; 1.3239x over previous
import functools

import jax
import jax.numpy as jnp
from jax import lax
from jax.experimental import pallas as pl
from jax.experimental.pallas import tpu as pltpu
from jax.experimental.pallas import tpu_sc as plsc

N_X = 4096
N_T = 4096
LANES = 128
SUB = N_T // LANES
R_TOTAL = N_X * SUB
BLK_X = 128
BLK_R = BLK_X * SUB

_SC_INFO = plsc.get_sparse_core_info()
_NC = _SC_INFO.num_cores
_NS = _SC_INFO.num_subcores
_NW = _NC * _NS
WORDS_PER_W = N_X * N_T // _NW
CHUNK_W = 65536
REP = CHUNK_W // N_T


def _bitonic_sort_2d(a):
    r_iota = jax.lax.broadcasted_iota(jnp.int32, (SUB, LANES), 0)
    c_iota = jax.lax.broadcasted_iota(jnp.int32, (SUB, LANES), 1)
    idx = r_iota * LANES + c_iota
    n = SUB * LANES
    k = 2
    while k <= n:
        j = k // 2
        while j >= 1:
            if j < LANES:
                fwd = jnp.roll(a, -j, axis=1)
                bwd = jnp.roll(a, j, axis=1)
            else:
                jr = j // LANES
                fwd = jnp.roll(a, -jr, axis=0)
                bwd = jnp.roll(a, jr, axis=0)
            lower = (idx & j) == 0
            p = jnp.where(lower, fwd, bwd)
            asc = (idx & k) == 0
            keep_min = lower == asc
            a = jnp.where(keep_min, jnp.minimum(a, p), jnp.maximum(a, p))
            j //= 2
        k *= 2
    return a


def _sort_kernel(t2d, ts_ref):
    ts_ref[:] = _bitonic_sort_2d(t2d[:])


def _xx_kernel(x_col, xx_ref):
    xb = x_col[:].reshape(BLK_X, 1, 1)
    xx_ref[:] = jnp.broadcast_to(xb, (BLK_X, SUB, LANES)).reshape(BLK_R, LANES)


_SC_MESH = plsc.VectorSubcoreMesh(core_axis_name="c", subcore_axis_name="s")


@functools.partial(
    pl.kernel,
    mesh=_SC_MESH,
    out_type=jax.ShapeDtypeStruct((N_X * N_T,), jnp.float32),
    scratch_types=[
        pltpu.VMEM_SHARED((CHUNK_W,), jnp.float32),
        pltpu.SemaphoreType.DMA,
        pltpu.SemaphoreType.DMA,
    ],
)
def _tt_sc_kernel(ts_hbm, out_hbm, shared, sem_r, sem_w):
    s = lax.axis_index("s")
    wid = s * _NC + lax.axis_index("c")
    base = wid * WORDS_PER_W

    @pl.when(s == 0)
    def _stage():
        reads = [
            pltpu.async_copy(ts_hbm, shared.at[pl.ds(r * N_T, N_T)], sem_r)
            for r in range(REP)
        ]
        for cp in reads:
            cp.wait()

    plsc.subcore_barrier()
    writes = [
        pltpu.async_copy(shared, out_hbm.at[pl.ds(base + ch * CHUNK_W, CHUNK_W)], sem_w)
        for ch in range(WORDS_PER_W // CHUNK_W)
    ]
    for cp in writes:
        cp.wait()


@jax.jit
def kernel(x_grid, t_grid):
    x_col = x_grid.reshape(N_X, 1)
    t2d = t_grid.reshape(SUB, LANES)

    ts2d = pl.pallas_call(
        _sort_kernel,
        out_shape=jax.ShapeDtypeStruct((SUB, LANES), jnp.float32),
    )(t2d)

    tt = _tt_sc_kernel(ts2d.reshape(N_T))

    xx = pl.pallas_call(
        _xx_kernel,
        grid=(N_X // BLK_X,),
        in_specs=[pl.BlockSpec((BLK_X, 1), lambda i: (i, 0))],
        out_specs=pl.BlockSpec((BLK_R, LANES), lambda i: (i, 0)),
        out_shape=jax.ShapeDtypeStruct((R_TOTAL, LANES), jnp.float32),
    )(x_col)

    return (xx.reshape(-1, 1), tt.reshape(-1, 1))


# speedup vs baseline: 1.7740x; 1.3400x over previous
import jax
import jax.numpy as jnp
from jax.experimental import pallas as pl
from jax.experimental.pallas import tpu as pltpu

N_X = 4096
N_T = 4096
LANES = 128
SUB = N_T // LANES
R_TOTAL = N_X * SUB
BLK_X = 256
BLK_R = BLK_X * SUB


def _bitonic_sort_2d(a):
    r_iota = jax.lax.broadcasted_iota(jnp.int32, (SUB, LANES), 0)
    c_iota = jax.lax.broadcasted_iota(jnp.int32, (SUB, LANES), 1)
    idx = r_iota * LANES + c_iota
    n = SUB * LANES
    k = 2
    while k <= n:
        j = k // 2
        while j >= 1:
            if j < LANES:
                fwd = jnp.roll(a, -j, axis=1)
                bwd = jnp.roll(a, j, axis=1)
            else:
                jr = j // LANES
                fwd = jnp.roll(a, -jr, axis=0)
                bwd = jnp.roll(a, jr, axis=0)
            lower = (idx & j) == 0
            p = jnp.where(lower, fwd, bwd)
            asc = (idx & k) == 0
            keep_min = lower == asc
            a = jnp.where(keep_min, jnp.minimum(a, p), jnp.maximum(a, p))
            j //= 2
        k *= 2
    return a


def _fused_kernel(x_col, t2d, xx_ref, tt_ref, ts2d):
    i = pl.program_id(0)

    @pl.when(i == 0)
    def _sort():
        ts2d[:] = _bitonic_sort_2d(t2d[:])

    xb = x_col[:].reshape(BLK_X, 1, 1)
    xx_ref[:] = jnp.broadcast_to(xb, (BLK_X, SUB, LANES)).reshape(BLK_R, LANES)
    ts = ts2d[:]
    tt_ref[:] = jnp.broadcast_to(ts[None], (BLK_X, SUB, LANES)).reshape(BLK_R, LANES)


@jax.jit
def kernel(x_grid, t_grid):
    x_col = x_grid.reshape(N_X, 1)
    t2d = t_grid.reshape(SUB, LANES)
    xx, tt = pl.pallas_call(
        _fused_kernel,
        grid=(N_X // BLK_X,),
        in_specs=[
            pl.BlockSpec((BLK_X, 1), lambda i: (i, 0)),
            pl.BlockSpec((SUB, LANES), lambda i: (0, 0)),
        ],
        out_specs=[
            pl.BlockSpec((BLK_R, LANES), lambda i: (i, 0)),
            pl.BlockSpec((BLK_R, LANES), lambda i: (i, 0)),
        ],
        out_shape=[
            jax.ShapeDtypeStruct((R_TOTAL, LANES), jnp.float32),
            jax.ShapeDtypeStruct((R_TOTAL, LANES), jnp.float32),
        ],
        scratch_shapes=[
            pltpu.VMEM((SUB, LANES), jnp.float32),
        ],
    )(x_col, t2d)
    return (xx.reshape(-1, 1), tt.reshape(-1, 1))
